# initial kernel scaffold (unmeasured)
import jax
import jax.numpy as jnp
from jax import lax
from jax.experimental import pallas as pl
from jax.experimental.pallas import tpu as pltpu

N_DEV = 4
NC = 4


def _gelu(y):
    c = 0.7978845608028654
    return 0.5 * y * (1.0 + jnp.tanh(c * (y + 0.044715 * y * y * y)))


def kernel(x, w_mat):
    partial = jnp.dot(x, w_mat, preferred_element_type=jnp.float32)
    return _allreduce_gelu(partial)


def _allreduce_gelu(partial):
    M, N = partial.shape
    BLK_M = M // N_DEV
    CH_M = BLK_M // NC

    def body(p_ref, out_ref, acc_hbm, recv_hbm, a_buf, b_buf,
             send_sems, recv_sems, credit_sem, cp_sem_a, cp_sem_b, cp_sem_w):
        my = lax.axis_index("i")
        left = (my + N_DEV - 1) % N_DEV
        right = (my + 1) % N_DEV

        barrier = pltpu.get_barrier_semaphore()
        for nbr in (left, right):
            pl.semaphore_signal(barrier, inc=1, device_id=(nbr,),
                                device_id_type=pl.DeviceIdType.MESH)
        pl.semaphore_wait(barrier, 2)

        it = 0

        for h in range(N_DEV - 1):
            sb = (my + N_DEV - h) % N_DEV
            rb = (my + N_DEV - h - 1) % N_DEV
            for c in range(NC):
                slot = it % 2
                if h == 0:
                    src = p_ref.at[pl.ds(sb * BLK_M + c * CH_M, CH_M), :]
                else:
                    src = acc_hbm.at[(h - 1) % 2, pl.ds(c * CH_M, CH_M), :]
                rdma = pltpu.make_async_remote_copy(
                    src_ref=src,
                    dst_ref=recv_hbm.at[slot],
                    send_sem=send_sems.at[slot],
                    recv_sem=recv_sems.at[slot],
                    device_id=(right,),
                    device_id_type=pl.DeviceIdType.MESH,
                )
                if it >= 2:
                    pl.semaphore_wait(credit_sem, 1)
                rdma.start()
                rdma.wait()
                cp_a = pltpu.make_async_copy(recv_hbm.at[slot], a_buf, cp_sem_a)
                cp_b = pltpu.make_async_copy(
                    p_ref.at[pl.ds(rb * BLK_M + c * CH_M, CH_M), :], b_buf,
                    cp_sem_b)
                cp_a.start()
                cp_b.start()
                cp_a.wait()
                cp_b.wait()
                if h < N_DEV - 2:
                    a_buf[:, :] = a_buf[:, :] + b_buf[:, :]
                    wb = pltpu.make_async_copy(
                        a_buf, acc_hbm.at[h % 2, pl.ds(c * CH_M, CH_M), :],
                        cp_sem_w)
                else:
                    a_buf[:, :] = _gelu(a_buf[:, :] + b_buf[:, :])
                    wb = pltpu.make_async_copy(
                        a_buf,
                        out_ref.at[pl.ds(rb * BLK_M + c * CH_M, CH_M), :],
                        cp_sem_w)
                wb.start()
                wb.wait()
                pl.semaphore_signal(credit_sem, inc=1, device_id=(left,),
                                    device_id_type=pl.DeviceIdType.MESH)
                it += 1

        for g in range(N_DEV - 1):
            gs = (my + N_DEV + 1 - g) % N_DEV
            for c in range(NC):
                slot = it % 2
                sl = pl.ds(gs * BLK_M + c * CH_M, CH_M)
                rdma = pltpu.make_async_remote_copy(
                    src_ref=out_ref.at[sl, :],
                    dst_ref=out_ref.at[sl, :],
                    send_sem=send_sems.at[slot],
                    recv_sem=recv_sems.at[slot],
                    device_id=(right,),
                    device_id_type=pl.DeviceIdType.MESH,
                )
                pl.semaphore_wait(credit_sem, 1)
                rdma.start()
                rdma.wait()
                pl.semaphore_signal(credit_sem, inc=1, device_id=(left,),
                                    device_id_type=pl.DeviceIdType.MESH)
                it += 1

        pl.semaphore_wait(credit_sem, 2)

    return pl.pallas_call(
        body,
        out_shape=jax.ShapeDtypeStruct((M, N), jnp.float32),
        in_specs=[pl.BlockSpec(memory_space=pl.ANY)],
        out_specs=pl.BlockSpec(memory_space=pl.ANY),
        scratch_shapes=[
            pltpu.HBM((2, BLK_M, N), jnp.float32),
            pltpu.HBM((2, CH_M, N), jnp.float32),
            pltpu.VMEM((CH_M, N), jnp.float32),
            pltpu.VMEM((CH_M, N), jnp.float32),
            pltpu.SemaphoreType.DMA((2,)),
            pltpu.SemaphoreType.DMA((2,)),
            pltpu.SemaphoreType.REGULAR,
            pltpu.SemaphoreType.DMA,
            pltpu.SemaphoreType.DMA,
            pltpu.SemaphoreType.DMA,
        ],
        compiler_params=pltpu.CompilerParams(collective_id=0),
    )(partial)


# baseline (device time: 2615509 ns/iter reference)
import jax
import jax.numpy as jnp
from jax import lax
from jax.experimental import pallas as pl
from jax.experimental.pallas import tpu as pltpu

N_DEV = 4
NC = 4


def _gelu(y):
    c = 0.7978845608028654
    return 0.5 * y * (1.0 + jnp.tanh(c * (y + 0.044715 * y * y * y)))


def kernel(x, w_mat):
    partial = jnp.dot(x, w_mat, preferred_element_type=jnp.float32)
    return _allreduce_gelu(partial)


def _allreduce_gelu(partial):
    M, N = partial.shape
    BLK_M = M // N_DEV
    CH_M = BLK_M // NC

    def body(p_ref, out_ref, acc_hbm, recv_hbm, a_buf, b_buf,
             send_sems, recv_sems, credit_sem, cp_sem_a, cp_sem_b, cp_sem_w):
        my = lax.axis_index("i")
        left = (my + N_DEV - 1) % N_DEV
        right = (my + 1) % N_DEV

        barrier = pltpu.get_barrier_semaphore()
        for nbr in (left, right):
            pl.semaphore_signal(barrier, inc=1, device_id=(nbr,),
                                device_id_type=pl.DeviceIdType.MESH)
        pl.semaphore_wait(barrier, 2)

        it = 0

        for h in range(N_DEV - 1):
            sb = (my + N_DEV - h) % N_DEV
            rb = (my + N_DEV - h - 1) % N_DEV
            for c in range(NC):
                slot = it % 2
                if h == 0:
                    src = p_ref.at[pl.ds(sb * BLK_M + c * CH_M, CH_M), :]
                else:
                    src = acc_hbm.at[(h - 1) % 2, pl.ds(c * CH_M, CH_M), :]
                rdma = pltpu.make_async_remote_copy(
                    src_ref=src,
                    dst_ref=recv_hbm.at[slot],
                    send_sem=send_sems.at[slot],
                    recv_sem=recv_sems.at[slot],
                    device_id=(right,),
                    device_id_type=pl.DeviceIdType.MESH,
                )
                if it >= 2:
                    pl.semaphore_wait(credit_sem, 1)
                rdma.start()
                rdma.wait()
                cp_a = pltpu.make_async_copy(recv_hbm.at[slot], a_buf, cp_sem_a)
                cp_b = pltpu.make_async_copy(
                    p_ref.at[pl.ds(rb * BLK_M + c * CH_M, CH_M), :], b_buf,
                    cp_sem_b)
                cp_a.start()
                cp_b.start()
                cp_a.wait()
                cp_b.wait()
                if h < N_DEV - 2:
                    a_buf[:, :] = a_buf[:, :] + b_buf[:, :]
                    wb = pltpu.make_async_copy(
                        a_buf, acc_hbm.at[h % 2, pl.ds(c * CH_M, CH_M), :],
                        cp_sem_w)
                else:
                    a_buf[:, :] = _gelu(a_buf[:, :] + b_buf[:, :])
                    wb = pltpu.make_async_copy(
                        a_buf,
                        out_ref.at[pl.ds(rb * BLK_M + c * CH_M, CH_M), :],
                        cp_sem_w)
                wb.start()
                wb.wait()
                pl.semaphore_signal(credit_sem, inc=1, device_id=(left,),
                                    device_id_type=pl.DeviceIdType.MESH)
                it += 1

        for g in range(N_DEV - 1):
            gs = (my + N_DEV + 1 - g) % N_DEV
            for c in range(NC):
                slot = it % 2
                sl = pl.ds(gs * BLK_M + c * CH_M, CH_M)
                rdma = pltpu.make_async_remote_copy(
                    src_ref=out_ref.at[sl, :],
                    dst_ref=out_ref.at[sl, :],
                    send_sem=send_sems.at[slot],
                    recv_sem=recv_sems.at[slot],
                    device_id=(right,),
                    device_id_type=pl.DeviceIdType.MESH,
                )
                pl.semaphore_wait(credit_sem, 1)
                rdma.start()
                rdma.wait()
                pl.semaphore_signal(credit_sem, inc=1, device_id=(left,),
                                    device_id_type=pl.DeviceIdType.MESH)
                it += 1

        pl.semaphore_wait(credit_sem, 2)

    out, _, _ = pl.pallas_call(
        body,
        out_shape=[
            jax.ShapeDtypeStruct((M, N), jnp.float32),
            jax.ShapeDtypeStruct((2, BLK_M, N), jnp.float32),
            jax.ShapeDtypeStruct((2, CH_M, N), jnp.float32),
        ],
        in_specs=[pl.BlockSpec(memory_space=pl.ANY)],
        out_specs=[
            pl.BlockSpec(memory_space=pl.ANY),
            pl.BlockSpec(memory_space=pl.ANY),
            pl.BlockSpec(memory_space=pl.ANY),
        ],
        scratch_shapes=[
            pltpu.VMEM((CH_M, N), jnp.float32),
            pltpu.VMEM((CH_M, N), jnp.float32),
            pltpu.SemaphoreType.DMA((2,)),
            pltpu.SemaphoreType.DMA((2,)),
            pltpu.SemaphoreType.REGULAR,
            pltpu.SemaphoreType.DMA,
            pltpu.SemaphoreType.DMA,
            pltpu.SemaphoreType.DMA,
        ],
        compiler_params=pltpu.CompilerParams(collective_id=0),
    )(partial)
    return out


# device time: 1621246 ns/iter; 1.6133x vs baseline; 1.6133x over previous
import jax
import jax.numpy as jnp
from jax import lax
from jax.experimental import pallas as pl
from jax.experimental.pallas import tpu as pltpu

N_DEV = 4
NC = 4


def _gelu(y):
    c = 0.7978845608028654
    return 0.5 * y * (1.0 + jnp.tanh(c * (y + 0.044715 * y * y * y)))


def kernel(x, w_mat):
    partial = jnp.dot(x, w_mat, preferred_element_type=jnp.float32)
    hn = partial.shape[1] // 2
    out_l, out_r = _allreduce_gelu(partial[:, :hn], partial[:, hn:])
    return jnp.concatenate([out_l, out_r], axis=1)


def _allreduce_gelu(p_l, p_r):
    M, HN = p_l.shape
    N = 2 * HN
    BLK_M = M // N_DEV
    CH_M = BLK_M // NC

    def body(pl_ref, pr_ref, outl_ref, outr_ref,
             acc_r_hbm, acc_l_hbm, recv_r_hbm, recv_l_hbm, a_buf, b_buf,
             send_sems_r, recv_sems_r, send_sems_l, recv_sems_l,
             credit_r, credit_l, cp_sems, wb_sems):
        my = lax.axis_index("i")
        left = (my + N_DEV - 1) % N_DEV
        right = (my + 1) % N_DEV

        barrier = pltpu.get_barrier_semaphore()
        for nbr in (left, right):
            pl.semaphore_signal(barrier, inc=1, device_id=(nbr,),
                                device_id_type=pl.DeviceIdType.MESH)
        pl.semaphore_wait(barrier, 2)

        it = 0

        for h in range(N_DEV - 1):
            sb_r = (my + N_DEV - h) % N_DEV
            rb_r = (my + N_DEV - h - 1) % N_DEV
            sb_l = (my + h) % N_DEV
            rb_l = (my + h + 1) % N_DEV
            for c in range(NC):
                slot = it % 2
                ro = c * CH_M
                if h == 0:
                    src_r = pl_ref.at[pl.ds(sb_r * BLK_M + ro, CH_M), :]
                    src_l = pr_ref.at[pl.ds(sb_l * BLK_M + ro, CH_M), :]
                else:
                    src_r = acc_r_hbm.at[(h - 1) % 2, pl.ds(ro, CH_M), :]
                    src_l = acc_l_hbm.at[(h - 1) % 2, pl.ds(ro, CH_M), :]
                rdma_r = pltpu.make_async_remote_copy(
                    src_ref=src_r,
                    dst_ref=recv_r_hbm.at[slot],
                    send_sem=send_sems_r.at[slot],
                    recv_sem=recv_sems_r.at[slot],
                    device_id=(right,),
                    device_id_type=pl.DeviceIdType.MESH,
                )
                rdma_l = pltpu.make_async_remote_copy(
                    src_ref=src_l,
                    dst_ref=recv_l_hbm.at[slot],
                    send_sem=send_sems_l.at[slot],
                    recv_sem=recv_sems_l.at[slot],
                    device_id=(left,),
                    device_id_type=pl.DeviceIdType.MESH,
                )
                if it >= 2:
                    pl.semaphore_wait(credit_r, 1)
                    pl.semaphore_wait(credit_l, 1)
                rdma_r.start()
                rdma_l.start()
                rdma_r.wait()
                rdma_l.wait()
                cps = [
                    pltpu.make_async_copy(
                        recv_r_hbm.at[slot], a_buf.at[:, pl.ds(0, HN)],
                        cp_sems.at[0]),
                    pltpu.make_async_copy(
                        recv_l_hbm.at[slot], a_buf.at[:, pl.ds(HN, HN)],
                        cp_sems.at[1]),
                    pltpu.make_async_copy(
                        pl_ref.at[pl.ds(rb_r * BLK_M + ro, CH_M), :],
                        b_buf.at[:, pl.ds(0, HN)], cp_sems.at[2]),
                    pltpu.make_async_copy(
                        pr_ref.at[pl.ds(rb_l * BLK_M + ro, CH_M), :],
                        b_buf.at[:, pl.ds(HN, HN)], cp_sems.at[3]),
                ]
                for cp in cps:
                    cp.start()
                for cp in cps:
                    cp.wait()
                if h < N_DEV - 2:
                    a_buf[:, :] = a_buf[:, :] + b_buf[:, :]
                    wbs = [
                        pltpu.make_async_copy(
                            a_buf.at[:, pl.ds(0, HN)],
                            acc_r_hbm.at[h % 2, pl.ds(ro, CH_M), :],
                            wb_sems.at[0]),
                        pltpu.make_async_copy(
                            a_buf.at[:, pl.ds(HN, HN)],
                            acc_l_hbm.at[h % 2, pl.ds(ro, CH_M), :],
                            wb_sems.at[1]),
                    ]
                else:
                    a_buf[:, :] = _gelu(a_buf[:, :] + b_buf[:, :])
                    wbs = [
                        pltpu.make_async_copy(
                            a_buf.at[:, pl.ds(0, HN)],
                            outl_ref.at[pl.ds(rb_r * BLK_M + ro, CH_M), :],
                            wb_sems.at[0]),
                        pltpu.make_async_copy(
                            a_buf.at[:, pl.ds(HN, HN)],
                            outr_ref.at[pl.ds(rb_l * BLK_M + ro, CH_M), :],
                            wb_sems.at[1]),
                    ]
                for wb in wbs:
                    wb.start()
                for wb in wbs:
                    wb.wait()
                pl.semaphore_signal(credit_r, inc=1, device_id=(left,),
                                    device_id_type=pl.DeviceIdType.MESH)
                pl.semaphore_signal(credit_l, inc=1, device_id=(right,),
                                    device_id_type=pl.DeviceIdType.MESH)
                it += 1

        for g in range(N_DEV - 1):
            gs_r = (my + N_DEV + 1 - g) % N_DEV
            gs_l = (my + N_DEV - 1 + g) % N_DEV
            for c in range(NC):
                slot = it % 2
                ro = c * CH_M
                sl_r = pl.ds(gs_r * BLK_M + ro, CH_M)
                sl_l = pl.ds(gs_l * BLK_M + ro, CH_M)
                rdma_r = pltpu.make_async_remote_copy(
                    src_ref=outl_ref.at[sl_r, :],
                    dst_ref=outl_ref.at[sl_r, :],
                    send_sem=send_sems_r.at[slot],
                    recv_sem=recv_sems_r.at[slot],
                    device_id=(right,),
                    device_id_type=pl.DeviceIdType.MESH,
                )
                rdma_l = pltpu.make_async_remote_copy(
                    src_ref=outr_ref.at[sl_l, :],
                    dst_ref=outr_ref.at[sl_l, :],
                    send_sem=send_sems_l.at[slot],
                    recv_sem=recv_sems_l.at[slot],
                    device_id=(left,),
                    device_id_type=pl.DeviceIdType.MESH,
                )
                pl.semaphore_wait(credit_r, 1)
                pl.semaphore_wait(credit_l, 1)
                rdma_r.start()
                rdma_l.start()
                rdma_r.wait()
                rdma_l.wait()
                pl.semaphore_signal(credit_r, inc=1, device_id=(left,),
                                    device_id_type=pl.DeviceIdType.MESH)
                pl.semaphore_signal(credit_l, inc=1, device_id=(right,),
                                    device_id_type=pl.DeviceIdType.MESH)
                it += 1

        pl.semaphore_wait(credit_r, 2)
        pl.semaphore_wait(credit_l, 2)

    out_l, out_r, *_ = pl.pallas_call(
        body,
        out_shape=[
            jax.ShapeDtypeStruct((M, HN), jnp.float32),
            jax.ShapeDtypeStruct((M, HN), jnp.float32),
            jax.ShapeDtypeStruct((2, BLK_M, HN), jnp.float32),
            jax.ShapeDtypeStruct((2, BLK_M, HN), jnp.float32),
            jax.ShapeDtypeStruct((2, CH_M, HN), jnp.float32),
            jax.ShapeDtypeStruct((2, CH_M, HN), jnp.float32),
        ],
        in_specs=[pl.BlockSpec(memory_space=pl.ANY)] * 2,
        out_specs=[pl.BlockSpec(memory_space=pl.ANY)] * 6,
        scratch_shapes=[
            pltpu.VMEM((CH_M, N), jnp.float32),
            pltpu.VMEM((CH_M, N), jnp.float32),
            pltpu.SemaphoreType.DMA((2,)),
            pltpu.SemaphoreType.DMA((2,)),
            pltpu.SemaphoreType.DMA((2,)),
            pltpu.SemaphoreType.DMA((2,)),
            pltpu.SemaphoreType.REGULAR,
            pltpu.SemaphoreType.REGULAR,
            pltpu.SemaphoreType.DMA((4,)),
            pltpu.SemaphoreType.DMA((2,)),
        ],
        compiler_params=pltpu.CompilerParams(collective_id=0),
    )(p_l, p_r)
    return out_l, out_r
